# Initial kernel scaffold; baseline (speedup 1.0000x reference)
import jax
import jax.numpy as jnp
from jax.experimental import pallas as pl


def kernel(L_u, L_i, user_w, item_w, user, pos_item, nega_item):
    raise NotImplementedError("write your pallas kernel here")


# speedup vs baseline: 3.6494x; 3.6494x over previous
import functools

import jax
import jax.numpy as jnp
from jax import lax
from jax.experimental import pallas as pl
from jax.experimental.pallas import tpu as pltpu
from jax.experimental.pallas import tpu_sc as plsc

N_ROWS = 8192
FEAT = 16
FEAT_P = 128
B = 4096
ROW_BLK = 256
L_W = 1e-4


def _proj_body(lu_ref, li_ref, uw_ref, iw_ref, u_ref, p_ref):
    lu = lu_ref[...].astype(jnp.bfloat16)
    li = li_ref[...].astype(jnp.bfloat16)
    uw = uw_ref[...].astype(jnp.bfloat16)
    iw = iw_ref[...].astype(jnp.bfloat16)
    u_ref[...] = jnp.dot(lu, uw, preferred_element_type=jnp.float32)
    p_ref[...] = jnp.dot(li, iw, preferred_element_type=jnp.float32)


def _project(L_u, L_i, user_w, item_w):
    grid = (N_ROWS // ROW_BLK,)
    return pl.pallas_call(
        _proj_body,
        grid=grid,
        in_specs=[
            pl.BlockSpec((ROW_BLK, N_ROWS), lambda i: (i, 0)),
            pl.BlockSpec((ROW_BLK, N_ROWS), lambda i: (i, 0)),
            pl.BlockSpec((N_ROWS, FEAT_P), lambda i: (0, 0)),
            pl.BlockSpec((N_ROWS, FEAT_P), lambda i: (0, 0)),
        ],
        out_specs=[
            pl.BlockSpec((ROW_BLK, FEAT_P), lambda i: (i, 0)),
            pl.BlockSpec((ROW_BLK, FEAT_P), lambda i: (i, 0)),
        ],
        out_shape=[
            jax.ShapeDtypeStruct((N_ROWS, FEAT_P), jnp.float32),
            jax.ShapeDtypeStruct((N_ROWS, FEAT_P), jnp.float32),
        ],
        compiler_params=pltpu.CompilerParams(
            dimension_semantics=("parallel",),
        ),
    )(L_u, L_i, user_w, item_w)


def _sc_gather(U, P, user, pos_item, nega_item):
    info = plsc.get_sparse_core_info()
    nc, ns = info.num_cores, info.num_subcores
    nw = nc * ns
    b_per_w = B // nw
    mesh = plsc.VectorSubcoreMesh(core_axis_name="c", subcore_axis_name="s")

    @functools.partial(
        pl.kernel,
        out_type=[
            jax.ShapeDtypeStruct((B, FEAT_P), jnp.float32),
            jax.ShapeDtypeStruct((B, FEAT_P), jnp.float32),
            jax.ShapeDtypeStruct((B, FEAT_P), jnp.float32),
        ],
        mesh=mesh,
        scratch_types=[
            pltpu.VMEM((b_per_w,), jnp.int32),
            pltpu.VMEM((b_per_w, FEAT_P), jnp.float32),
            pltpu.SemaphoreType.DMA,
        ],
    )
    def gather_kernel(u_hbm, p_hbm, uidx_hbm, pidx_hbm, nidx_hbm,
                      ou_hbm, op_hbm, on_hbm, idx_v, rows_v, sem):
        wid = lax.axis_index("s") * nc + lax.axis_index("c")
        base = wid * b_per_w
        for tbl, idx, out in ((u_hbm, uidx_hbm, ou_hbm),
                              (p_hbm, pidx_hbm, op_hbm),
                              (p_hbm, nidx_hbm, on_hbm)):
            pltpu.sync_copy(idx.at[pl.ds(base, b_per_w)], idx_v)
            pltpu.async_copy(tbl.at[idx_v], rows_v, sem).wait()
            pltpu.sync_copy(rows_v, out.at[pl.ds(base, b_per_w)])

    return gather_kernel(U, P, user, pos_item, nega_item)


def _loss_body(gu_ref, gp_ref, gn_ref, out_ref):
    gu = gu_ref[...]
    gp = gp_ref[...]
    gn = gn_ref[...]
    s_pos = jnp.sum(gu * gp, axis=1)
    s_neg = jnp.sum(gu * gn, axis=1)
    out = jax.nn.sigmoid(s_pos - s_neg)
    reg = L_W * (jnp.sum(gu * gu) + jnp.sum(gp * gp) + jnp.sum(gn * gn))
    loss = (-jnp.sum(jnp.log(out + 1e-8)) + reg) / B
    out_ref[...] = jnp.reshape(loss, (1, 1))


def _loss(gu, gp, gn):
    return pl.pallas_call(
        _loss_body,
        out_shape=jax.ShapeDtypeStruct((1, 1), jnp.float32),
    )(gu, gp, gn)


def kernel(L_u, L_i, user_w, item_w, user, pos_item, nega_item):
    user_w = jnp.pad(user_w, ((0, 0), (0, FEAT_P - FEAT)))
    item_w = jnp.pad(item_w, ((0, 0), (0, FEAT_P - FEAT)))
    U, P = _project(L_u, L_i, user_w, item_w)
    gu, gp, gn = _sc_gather(U, P, user, pos_item, nega_item)
    return _loss(gu, gp, gn)[0, 0]


# speedup vs baseline: 3.7733x; 1.0339x over previous
import functools

import jax
import jax.numpy as jnp
from jax import lax
from jax.experimental import pallas as pl
from jax.experimental.pallas import tpu as pltpu
from jax.experimental.pallas import tpu_sc as plsc

N_ROWS = 8192
FEAT = 16
FEAT_P = 128
B = 4096
ROW_BLK = 256
L_W = 1e-4


def _proj_body(lu_ref, li_ref, uw_ref, iw_ref, u_ref, p_ref):
    lu = lu_ref[...].astype(jnp.bfloat16)
    li = li_ref[...].astype(jnp.bfloat16)
    u = jnp.dot(lu, uw_ref[...].astype(jnp.bfloat16),
                preferred_element_type=jnp.float32)
    p = jnp.dot(li, iw_ref[...].astype(jnp.bfloat16),
                preferred_element_type=jnp.float32)
    u_ref[...] = jnp.pad(u, ((0, 0), (0, FEAT_P - FEAT)))
    p_ref[...] = jnp.pad(p, ((0, 0), (0, FEAT_P - FEAT)))


def _project(L_u, L_i, user_w, item_w):
    grid = (N_ROWS // ROW_BLK,)
    return pl.pallas_call(
        _proj_body,
        grid=grid,
        in_specs=[
            pl.BlockSpec((ROW_BLK, N_ROWS), lambda i: (i, 0)),
            pl.BlockSpec((ROW_BLK, N_ROWS), lambda i: (i, 0)),
            pl.BlockSpec((N_ROWS, FEAT), lambda i: (0, 0)),
            pl.BlockSpec((N_ROWS, FEAT), lambda i: (0, 0)),
        ],
        out_specs=[
            pl.BlockSpec((ROW_BLK, FEAT_P), lambda i: (i, 0)),
            pl.BlockSpec((ROW_BLK, FEAT_P), lambda i: (i, 0)),
        ],
        out_shape=[
            jax.ShapeDtypeStruct((N_ROWS, FEAT_P), jnp.float32),
            jax.ShapeDtypeStruct((N_ROWS, FEAT_P), jnp.float32),
        ],
        compiler_params=pltpu.CompilerParams(
            dimension_semantics=("parallel",),
        ),
    )(L_u, L_i, user_w, item_w)


def _sc_gather(U, P, user, pos_item, nega_item):
    info = plsc.get_sparse_core_info()
    nc, ns = info.num_cores, info.num_subcores
    nw = nc * ns
    b_per_w = B // nw
    mesh = plsc.VectorSubcoreMesh(core_axis_name="c", subcore_axis_name="s")

    @functools.partial(
        pl.kernel,
        out_type=jax.ShapeDtypeStruct((3 * B, FEAT_P), jnp.float32),
        mesh=mesh,
        scratch_types=[
            pltpu.VMEM((b_per_w,), jnp.int32),
            pltpu.VMEM((b_per_w,), jnp.int32),
            pltpu.VMEM((b_per_w,), jnp.int32),
            pltpu.VMEM((b_per_w, FEAT_P), jnp.float32),
            pltpu.VMEM((b_per_w, FEAT_P), jnp.float32),
            pltpu.VMEM((b_per_w, FEAT_P), jnp.float32),
            pltpu.SemaphoreType.DMA,
        ],
    )
    def gather_kernel(u_hbm, p_hbm, uidx_hbm, pidx_hbm, nidx_hbm,
                      out_hbm, idx_u, idx_p, idx_n,
                      rows_u, rows_p, rows_n, sem):
        wid = lax.axis_index("s") * nc + lax.axis_index("c")
        base = wid * b_per_w
        sl = pl.ds(base, b_per_w)
        pltpu.sync_copy(uidx_hbm.at[sl], idx_u)
        pltpu.sync_copy(pidx_hbm.at[sl], idx_p)
        pltpu.sync_copy(nidx_hbm.at[sl], idx_n)
        c1 = pltpu.async_copy(u_hbm.at[idx_u], rows_u, sem)
        c2 = pltpu.async_copy(p_hbm.at[idx_p], rows_p, sem)
        c3 = pltpu.async_copy(p_hbm.at[idx_n], rows_n, sem)
        c1.wait()
        c2.wait()
        c3.wait()
        pltpu.sync_copy(rows_u, out_hbm.at[pl.ds(base, b_per_w)])
        pltpu.sync_copy(rows_p, out_hbm.at[pl.ds(B + base, b_per_w)])
        pltpu.sync_copy(rows_n, out_hbm.at[pl.ds(2 * B + base, b_per_w)])

    return gather_kernel(U, P, user, pos_item, nega_item)


def _loss_body(g_ref, out_ref):
    gu = g_ref[0:B, :]
    gp = g_ref[B:2 * B, :]
    gn = g_ref[2 * B:3 * B, :]
    s_pos = jnp.sum(gu * gp, axis=1)
    s_neg = jnp.sum(gu * gn, axis=1)
    out = jax.nn.sigmoid(s_pos - s_neg)
    reg = L_W * (jnp.sum(gu * gu) + jnp.sum(gp * gp) + jnp.sum(gn * gn))
    loss = (-jnp.sum(jnp.log(out + 1e-8)) + reg) / B
    out_ref[...] = jnp.reshape(loss, (1, 1))


def _loss(g):
    return pl.pallas_call(
        _loss_body,
        out_shape=jax.ShapeDtypeStruct((1, 1), jnp.float32),
    )(g)


def kernel(L_u, L_i, user_w, item_w, user, pos_item, nega_item):
    U, P = _project(L_u, L_i, user_w, item_w)
    g = _sc_gather(U, P, user, pos_item, nega_item)
    return _loss(g)[0, 0]
